# baseline (speedup 1.0000x reference)
import jax
import jax.numpy as jnp
from jax.experimental import pallas as pl
from jax.experimental.pallas import tpu as pltpu


def _gcn_body(adj_ref, nodes_ref, w_ref, out_ref):
    a = adj_ref[...].astype(jnp.bfloat16)
    x = nodes_ref[...].astype(jnp.bfloat16)
    agg = jax.lax.dot_general(
        a, x, (((2,), (1,)), ((0,), (0,))),
        preferred_element_type=jnp.float32).astype(jnp.bfloat16)
    w = w_ref[...].astype(jnp.bfloat16)
    out_ref[...] = jax.lax.dot_general(
        agg, w, (((2,), (1,)), ((0,), (0,))),
        preferred_element_type=jnp.float32)


def kernel(adjacency, nodes, weights):
    adjacency = adjacency.astype(jnp.float32)
    nodes = nodes.astype(jnp.float32)
    weights = weights.astype(jnp.float32)

    B, W, N, _ = adjacency.shape
    Fin = nodes.shape[-1]
    Wp, _, Fout = weights.shape
    w_used = weights[Wp - W:, :, :]

    G = 2 * W
    S = (B * W) // G
    adj_f = adjacency.reshape(S, G, N, N)
    nodes_f = nodes.reshape(S, G, N, Fin)
    w_f = jnp.broadcast_to(w_used[None], (2, W, Fin, Fout)).reshape(G, Fin, Fout)

    out = pl.pallas_call(
        _gcn_body,
        grid=(S,),
        in_specs=[
            pl.BlockSpec((None, G, N, N), lambda s: (s, 0, 0, 0)),
            pl.BlockSpec((None, G, N, Fin), lambda s: (s, 0, 0, 0)),
            pl.BlockSpec((G, Fin, Fout), lambda s: (0, 0, 0)),
        ],
        out_specs=pl.BlockSpec((None, G, N, Fout), lambda s: (s, 0, 0, 0)),
        out_shape=jax.ShapeDtypeStruct((S, G, N, Fout), jnp.float32),
        compiler_params=pltpu.CompilerParams(
            dimension_semantics=("parallel",)),
    )(adj_f, nodes_f, w_f)
    return out.reshape(B, W, N, Fout)
